# baseline (speedup 1.0000x reference)
import functools

import jax
import jax.numpy as jnp
from jax import lax
from jax.experimental import pallas as pl
from jax.experimental.pallas import tpu as pltpu
from jax.experimental.pallas import tpu_sc as plsc

NTOKEN = 100000
EMB_DIM = 64
BATCH = 4096
HIST = 50
TOT = BATCH * HIST

_info = plsc.get_sparse_core_info()
NC = _info.num_cores
NS = _info.num_subcores
NW = NC * NS
BPW = TOT // NW
CHUNK = 800
NCHUNK = BPW // CHUNK

_mesh = plsc.VectorSubcoreMesh(core_axis_name="c", subcore_axis_name="s")


@functools.partial(
    pl.kernel,
    mesh=_mesh,
    out_type=jax.ShapeDtypeStruct((TOT, EMB_DIM), jnp.float32),
    scratch_types=[
        pltpu.VMEM((CHUNK,), jnp.int32),
        pltpu.VMEM((CHUNK,), jnp.int32),
        pltpu.VMEM((CHUNK, EMB_DIM), jnp.float32),
        pltpu.VMEM((CHUNK, EMB_DIM), jnp.float32),
        pltpu.SemaphoreType.DMA,
        pltpu.SemaphoreType.DMA,
        pltpu.SemaphoreType.DMA,
        pltpu.SemaphoreType.DMA,
    ],
    compiler_params=pltpu.CompilerParams(use_tc_tiling_on_sc=False),
)
def _gather_kernel(idx_hbm, table_hbm, out_hbm, i0, i1, r0, r1, gs0, gs1, os0, os1):
    wid = lax.axis_index("s") * NC + lax.axis_index("c")
    base = wid * BPW
    ib, rb, gs, os_ = [i0, i1], [r0, r1], [gs0, gs1], [os0, os1]

    def off(i):
        return base + i * CHUNK

    gathers = [None] * NCHUNK
    outs = [None] * NCHUNK
    for i in range(min(2, NCHUNK)):
        b = i % 2
        pltpu.sync_copy(idx_hbm.at[pl.ds(off(i), CHUNK)], ib[b])
        gathers[i] = pltpu.async_copy(table_hbm.at[ib[b]], rb[b], gs[b])
    for i in range(NCHUNK):
        b = i % 2
        gathers[i].wait()
        outs[i] = pltpu.async_copy(rb[b], out_hbm.at[pl.ds(off(i), CHUNK)], os_[b])
        if i + 2 < NCHUNK:
            outs[i].wait()
            pltpu.sync_copy(idx_hbm.at[pl.ds(off(i + 2), CHUNK)], ib[b])
            gathers[i + 2] = pltpu.async_copy(table_hbm.at[ib[b]], rb[b], gs[b])
    outs[NCHUNK - 2].wait()
    outs[NCHUNK - 1].wait()


def kernel(x, table):
    flat = x.reshape(TOT).astype(jnp.int32)
    out = _gather_kernel(flat, table)
    return out.reshape(BATCH, HIST, EMB_DIM)
